# baseline (speedup 1.0000x reference)
import functools

import jax
import jax.numpy as jnp
from jax import lax
from jax.experimental import pallas as pl
from jax.experimental.pallas import tpu as pltpu
from jax.experimental.pallas import tpu_sc as plsc

_B, _L, _K, _D, _OUT = 1024, 50, 5, 128, 8
_PAD = 1
_NC, _NS = 2, 16
_NW = _NC * _NS
_NIDX = _B * _L
_PERW = _NIDX // _NW
_CHUNKS = [128] * 12 + [64]

_BB = 128
_NB = _B // _BB
_NEG = -1e18


def _ir_gather_sc(table, idx_flat):
    mesh = plsc.VectorSubcoreMesh(core_axis_name="c", subcore_axis_name="s")

    @functools.partial(
        pl.kernel,
        mesh=mesh,
        out_type=jax.ShapeDtypeStruct((_NIDX,), jnp.float32),
        scratch_types=[
            pltpu.VMEM((_PERW,), jnp.int32),
            pltpu.VMEM((_PERW,), jnp.float32),
            pltpu.SemaphoreType.DMA,
        ],
    )
    def gather_kernel(table_hbm, idx_hbm, out_hbm, idx_v, vals_v, sem):
        wid = lax.axis_index("s") * _NC + lax.axis_index("c")
        base = wid * _PERW
        pltpu.sync_copy(idx_hbm.at[pl.ds(base, _PERW)], idx_v)
        copies = []
        off = 0
        for n in _CHUNKS:
            copies.append(pltpu.async_copy(
                table_hbm.at[idx_v.at[pl.ds(off, n)]],
                vals_v.at[pl.ds(off, n)],
                sem,
            ))
            off += n
        for c in copies:
            c.wait()
        pltpu.sync_copy(vals_v, out_hbm.at[pl.ds(base, _PERW)])

    return gather_kernel(table, idx_flat)


def _stream_body(x_ref, ew_ref, nbr_ref, w_ref, u_ref, p_ref):
    m = None
    for k in range(_K):
        t = ew_ref[k][:, :, None] * nbr_ref[:, k]
        t = jnp.where(t == 0.0, _NEG, t)
        m = t if m is None else jnp.maximum(m, t)
    sm = jnp.sum(m, axis=0)
    w = w_ref[...]
    u_ref[...] = lax.dot_general(w, sm, (((1,), (1,)), ((), ())),
                                 preferred_element_type=jnp.float32,
                                 precision=lax.Precision.HIGHEST)
    a = (x_ref[...] - m).reshape(_L * _BB, _D)
    p_ref[...] = lax.dot_general(w, a, (((1,), (1,)), ((), ())),
                                 preferred_element_type=jnp.float32,
                                 precision=lax.Precision.HIGHEST)


def _combine_body(u_ref, p_ref, ns_ref, ir_ref, b_ref, out_ref):
    g = jnp.where(ns_ref[...] == _PAD, 1.0, ir_ref[...])
    u = u_ref[...]
    bcol = jnp.transpose(b_ref[...], (1, 0))
    for i in range(_NB):
        s2 = None
        for l in range(_L):
            t = g[l * _NB + i:l * _NB + i + 1, :] * \
                p_ref[:, i * (_L * _BB) + l * _BB:i * (_L * _BB) + (l + 1) * _BB]
            s2 = t if s2 is None else s2 + t
        z = u[:, i * _BB:(i + 1) * _BB] + s2 + bcol
        z = jnp.maximum(z, 0.0)
        z = z - jnp.max(z, axis=0, keepdims=True)
        e = jnp.exp(z)
        out_ref[:, i * _BB:(i + 1) * _BB] = e / jnp.sum(e, axis=0, keepdims=True)


def _tc_stream(x_t, ew_t, nbr_t, W):
    return pl.pallas_call(
        _stream_body,
        grid=(_NB,),
        in_specs=[
            pl.BlockSpec((_L, _BB, _D), lambda i: (0, i, 0)),
            pl.BlockSpec((_K, _L, _BB), lambda i: (0, 0, i)),
            pl.BlockSpec((_L, _K, _BB, _D), lambda i: (0, 0, i, 0)),
            pl.BlockSpec((_OUT, _D), lambda i: (0, 0)),
        ],
        out_specs=[
            pl.BlockSpec((_OUT, _BB), lambda i: (0, i)),
            pl.BlockSpec((_OUT, _L * _BB), lambda i: (0, i)),
        ],
        out_shape=[
            jax.ShapeDtypeStruct((_OUT, _B), jnp.float32),
            jax.ShapeDtypeStruct((_OUT, _NIDX), jnp.float32),
        ],
    )(x_t, ew_t, nbr_t, W)


def _tc_combine(u, p, ns400, ir400, b2):
    return pl.pallas_call(
        _combine_body,
        grid=(1,),
        in_specs=[
            pl.BlockSpec((_OUT, _B), lambda i: (0, 0)),
            pl.BlockSpec((_OUT, _NIDX), lambda i: (0, 0)),
            pl.BlockSpec((_NIDX // 128, 128), lambda i: (0, 0)),
            pl.BlockSpec((_NIDX // 128, 128), lambda i: (0, 0)),
            pl.BlockSpec((1, _OUT), lambda i: (0, 0)),
        ],
        out_specs=pl.BlockSpec((_OUT, _B), lambda i: (0, 0)),
        out_shape=jax.ShapeDtypeStruct((_OUT, _B), jnp.float32),
    )(u, p, ns400, ir400, b2)


def kernel(node_sets, embedded_node, edge_weight, embedded_neighbor_node,
           information_rate, W, b):
    ns_t = jnp.transpose(jnp.asarray(node_sets, jnp.int32), (1, 0))
    x_t = jnp.transpose(embedded_node, (1, 0, 2))
    ew_t = jnp.transpose(edge_weight, (2, 1, 0))
    nbr_t = jnp.transpose(embedded_neighbor_node, (1, 2, 0, 3))
    table = information_rate.reshape(-1)
    ns_flat = ns_t.reshape(_NIDX)
    b2 = b.reshape(1, _OUT)
    u, p = _tc_stream(x_t, ew_t, nbr_t, W)
    ir_flat = _ir_gather_sc(table, ns_flat)
    out_t = _tc_combine(u, p, ns_flat.reshape(_NIDX // 128, 128),
                        ir_flat.reshape(_NIDX // 128, 128), b2)
    return jnp.transpose(out_t, (1, 0))


# speedup vs baseline: 1.0489x; 1.0489x over previous
import functools

import jax
import jax.numpy as jnp
from jax import lax
from jax.experimental import pallas as pl
from jax.experimental.pallas import tpu as pltpu
from jax.experimental.pallas import tpu_sc as plsc

_B, _L, _K, _D, _OUT = 1024, 50, 5, 128, 8
_PAD = 1
_NC, _NS = 2, 16
_NW = _NC * _NS
_NIDX = _B * _L
_PERW = _NIDX // _NW
_CHUNKS = [128] * 12 + [64]

_BB = 128
_NB = _B // _BB
_NEG = -1e18


def _ir_gather_sc(table, idx_flat):
    mesh = plsc.VectorSubcoreMesh(core_axis_name="c", subcore_axis_name="s")

    @functools.partial(
        pl.kernel,
        mesh=mesh,
        out_type=jax.ShapeDtypeStruct((_NIDX,), jnp.float32),
        scratch_types=[
            pltpu.VMEM((_PERW,), jnp.int32),
            pltpu.VMEM((_PERW,), jnp.float32),
            pltpu.SemaphoreType.DMA,
        ],
    )
    def gather_kernel(table_hbm, idx_hbm, out_hbm, idx_v, vals_v, sem):
        wid = lax.axis_index("s") * _NC + lax.axis_index("c")
        base = wid * _PERW
        pltpu.sync_copy(idx_hbm.at[pl.ds(base, _PERW)], idx_v)
        copies = []
        off = 0
        for n in _CHUNKS:
            copies.append(pltpu.async_copy(
                table_hbm.at[idx_v.at[pl.ds(off, n)]],
                vals_v.at[pl.ds(off, n)],
                sem,
            ))
            off += n
        for c in copies:
            c.wait()
        pltpu.sync_copy(vals_v, out_hbm.at[pl.ds(base, _PERW)])

    return gather_kernel(table, idx_flat)


def _tc_body(ns_ref, x_ref, ew_ref, nbr_ref, ir_ref, w_ref, b_ref, out_ref):
    m = None
    for k in range(_K):
        t = ew_ref[k][:, :, None] * nbr_ref[:, k]
        t = jnp.where(t == 0.0, _NEG, t)
        m = t if m is None else jnp.maximum(m, t)
    g = jnp.where(ns_ref[...] == _PAD, 1.0, ir_ref[...])
    gb = g[:, :, None]
    emb = (1.0 - gb) * m + gb * x_ref[...]
    s = jnp.sum(emb, axis=0)
    z = lax.dot_general(s, w_ref[...], (((1,), (1,)), ((), ())),
                        preferred_element_type=jnp.float32)
    z = jnp.maximum(z + b_ref[...], 0.0)
    z = z - jnp.max(z, axis=1, keepdims=True)
    e = jnp.exp(z)
    out_ref[...] = e / jnp.sum(e, axis=1, keepdims=True)


def _tc_call(ns_t, x_t, ew_t, nbr_t, ir_t, W, b2):
    return pl.pallas_call(
        _tc_body,
        grid=(_NB,),
        in_specs=[
            pl.BlockSpec((_L, _BB), lambda i: (0, i)),
            pl.BlockSpec((_L, _BB, _D), lambda i: (0, i, 0)),
            pl.BlockSpec((_K, _L, _BB), lambda i: (0, 0, i)),
            pl.BlockSpec((_L, _K, _BB, _D), lambda i: (0, 0, i, 0)),
            pl.BlockSpec((_L, _BB), lambda i: (0, i)),
            pl.BlockSpec((_OUT, _D), lambda i: (0, 0)),
            pl.BlockSpec((1, _OUT), lambda i: (0, 0)),
        ],
        out_specs=pl.BlockSpec((_BB, _OUT), lambda i: (i, 0)),
        out_shape=jax.ShapeDtypeStruct((_B, _OUT), jnp.float32),
    )(ns_t, x_t, ew_t, nbr_t, ir_t, W, b2)


def kernel(node_sets, embedded_node, edge_weight, embedded_neighbor_node,
           information_rate, W, b):
    ns_t = jnp.transpose(jnp.asarray(node_sets, jnp.int32), (1, 0))
    x_t = jnp.transpose(embedded_node, (1, 0, 2))
    ew_t = jnp.transpose(edge_weight, (2, 1, 0))
    nbr_t = jnp.transpose(embedded_neighbor_node, (1, 2, 0, 3))
    table = information_rate.reshape(-1)
    ns_flat = ns_t.reshape(_NIDX)
    b2 = b.reshape(1, _OUT)
    ir_t = _ir_gather_sc(table, ns_flat).reshape(_L, _B)
    return _tc_call(ns_t, x_t, ew_t, nbr_t, ir_t, W, b2)


# speedup vs baseline: 1.0727x; 1.0227x over previous
import functools

import jax
import jax.numpy as jnp
from jax import lax
from jax.experimental import pallas as pl
from jax.experimental.pallas import tpu as pltpu
from jax.experimental.pallas import tpu_sc as plsc

_B, _L, _K, _D, _OUT = 1024, 50, 5, 128, 8
_PAD = 1
_NC, _NS = 2, 16
_NW = _NC * _NS
_NIDX = _B * _L
_PERW = _NIDX // _NW
_CHUNKS = [128] * 12 + [64]

_BB = 128
_NB = _B // _BB
_NEG = -1e18


def _ir_gather_sc(table, idx_flat):
    mesh = plsc.VectorSubcoreMesh(core_axis_name="c", subcore_axis_name="s")

    @functools.partial(
        pl.kernel,
        mesh=mesh,
        out_type=jax.ShapeDtypeStruct((_NIDX,), jnp.float32),
        scratch_types=[
            pltpu.VMEM((_PERW,), jnp.int32),
            pltpu.VMEM((_PERW,), jnp.float32),
            pltpu.SemaphoreType.DMA,
        ],
    )
    def gather_kernel(table_hbm, idx_hbm, out_hbm, idx_v, vals_v, sem):
        wid = lax.axis_index("s") * _NC + lax.axis_index("c")
        base = wid * _PERW
        pltpu.sync_copy(idx_hbm.at[pl.ds(base, _PERW)], idx_v)
        copies = []
        off = 0
        for n in _CHUNKS:
            copies.append(pltpu.async_copy(
                table_hbm.at[idx_v.at[pl.ds(off, n)]],
                vals_v.at[pl.ds(off, n)],
                sem,
            ))
            off += n
        for c in copies:
            c.wait()
        pltpu.sync_copy(vals_v, out_hbm.at[pl.ds(base, _PERW)])

    return gather_kernel(table, idx_flat)


def _tc_body(ns_ref, x_ref, ew_ref, nbr_ref, ir_ref, w_ref, b_ref, out_ref):
    m = None
    for k in range(_K):
        t = ew_ref[k][:, :, None] * nbr_ref[:, k]
        t = jnp.where(t == 0.0, _NEG, t)
        m = t if m is None else jnp.maximum(m, t)
    g = jnp.where(ns_ref[...] == _PAD, 1.0, ir_ref[...])
    gb = g[:, :, None]
    emb = (1.0 - gb) * m + gb * x_ref[...]
    s = jnp.sum(emb, axis=0)
    z = lax.dot_general(s, w_ref[...], (((1,), (1,)), ((), ())),
                        preferred_element_type=jnp.float32)
    z = jnp.maximum(z + b_ref[...], 0.0)
    z = z - jnp.max(z, axis=1, keepdims=True)
    e = jnp.exp(z)
    y = e / jnp.sum(e, axis=1, keepdims=True)
    out_ref[...] = jnp.transpose(y, (1, 0))


def _tc_call(ns_t, x_t, ew_t, nbr_t, ir_t, W, b2):
    return pl.pallas_call(
        _tc_body,
        grid=(_NB,),
        in_specs=[
            pl.BlockSpec((_L, _BB), lambda i: (0, i)),
            pl.BlockSpec((_L, _BB, _D), lambda i: (0, i, 0)),
            pl.BlockSpec((_K, _L, _BB), lambda i: (0, 0, i)),
            pl.BlockSpec((_L, _K, _BB, _D), lambda i: (0, 0, i, 0)),
            pl.BlockSpec((_L, _BB), lambda i: (0, i)),
            pl.BlockSpec((_OUT, _D), lambda i: (0, 0)),
            pl.BlockSpec((1, _OUT), lambda i: (0, 0)),
        ],
        out_specs=pl.BlockSpec((_OUT, _BB), lambda i: (0, i)),
        out_shape=jax.ShapeDtypeStruct((_OUT, _B), jnp.float32),
    )(ns_t, x_t, ew_t, nbr_t, ir_t, W, b2)


def kernel(node_sets, embedded_node, edge_weight, embedded_neighbor_node,
           information_rate, W, b):
    ns_t = jnp.transpose(jnp.asarray(node_sets, jnp.int32), (1, 0))
    x_t = jnp.transpose(embedded_node, (1, 0, 2))
    ew_t = jnp.transpose(edge_weight, (2, 1, 0))
    nbr_t = jnp.transpose(embedded_neighbor_node, (1, 2, 0, 3))
    table = information_rate.reshape(-1)
    ns_flat = ns_t.reshape(_NIDX)
    b2 = b.reshape(1, _OUT)
    ir_t = _ir_gather_sc(table, ns_flat).reshape(_L, _B)
    out_t = _tc_call(ns_t, x_t, ew_t, nbr_t, ir_t, W, b2)
    return jnp.transpose(out_t, (1, 0))
